# baseline (speedup 1.0000x reference)
import functools

import jax
import jax.numpy as jnp
from jax import lax
from jax.experimental import pallas as pl
from jax.experimental.pallas import tpu as pltpu
from jax.experimental.pallas import tpu_sc as plsc

NUM_STATES = 128
HALF_STATES = NUM_STATES // 2
NUM_OBS = 100000
BATCH = 16384

COL_BLK = 8192
HALF_BLK = COL_BLK // 2
NBLK = (NUM_OBS + COL_BLK - 1) // COL_BLK
TBL_ROWS = NBLK * HALF_BLK

NC = 2
NS = 16
NW = NC * NS
BPW = BATCH // NW
IDX_ROWS = BPW // 128

EPI_BLK = 2048
EPI_NBLK = BATCH // EPI_BLK


def _round_bits(x):
    b = lax.bitcast_convert_type(x, jnp.uint32)
    return (b + jnp.uint32(0x8000)) & jnp.uint32(0xFFFF0000)


def _lse_transpose_pack_body(x_ref, tpk_ref, lse_ref, m_acc, s_acc):
    i = pl.program_id(0)
    xt = x_ref[...].T

    lo_bits = _round_bits(xt[:, :HALF_STATES])
    hi_bits = _round_bits(xt[:, HALF_STATES:])
    pk = (lo_bits >> 16) | hi_bits
    tpk_ref[...] = lax.bitcast_convert_type(
        jnp.concatenate([pk[:HALF_BLK], pk[HALF_BLK:]], axis=1), jnp.int32)

    @pl.when(i == 0)
    def _():
        m_acc[...] = jnp.full((1, NUM_STATES), -jnp.inf, jnp.float32)
        s_acc[...] = jnp.zeros((1, NUM_STATES), jnp.float32)

    def update(xt_vals):
        m_old = m_acc[...]
        m_new = jnp.maximum(m_old, jnp.max(xt_vals, axis=0, keepdims=True))
        s_acc[...] = s_acc[...] * jnp.exp(m_old - m_new) + jnp.sum(
            jnp.exp(xt_vals - m_new), axis=0, keepdims=True)
        m_acc[...] = m_new

    @pl.when(i < NBLK - 1)
    def _():
        update(xt)

    @pl.when(i == NBLK - 1)
    def _():
        row = i * COL_BLK + lax.broadcasted_iota(
            jnp.int32, (COL_BLK, NUM_STATES), 0)
        update(jnp.where(row < NUM_OBS, xt, -jnp.inf))
        lse_ref[...] = m_acc[...] + jnp.log(s_acc[...])


_tc_pass = pl.pallas_call(
    _lse_transpose_pack_body,
    grid=(NBLK,),
    in_specs=[pl.BlockSpec((NUM_STATES, COL_BLK), lambda i: (0, i))],
    out_specs=[
        pl.BlockSpec((HALF_BLK, NUM_STATES), lambda i: (i, 0)),
        pl.BlockSpec((1, NUM_STATES), lambda i: (0, 0)),
    ],
    out_shape=[
        jax.ShapeDtypeStruct((TBL_ROWS, NUM_STATES), jnp.int32),
        jax.ShapeDtypeStruct((1, NUM_STATES), jnp.float32),
    ],
    scratch_shapes=[
        pltpu.VMEM((1, NUM_STATES), jnp.float32),
        pltpu.VMEM((1, NUM_STATES), jnp.float32),
    ],
    compiler_params=pltpu.CompilerParams(
        dimension_semantics=("arbitrary",)),
)


def _sc_gather_body(table_hbm, obs_hbm, raw_hbm, idx_v, rows_v, sem):
    wid = lax.axis_index("s") * NC + lax.axis_index("c")
    base = wid * BPW
    pltpu.sync_copy(obs_hbm.at[wid], idx_v)
    copies = [
        pltpu.async_copy(table_hbm.at[idx_v.at[j]],
                         rows_v.at[pl.ds(j * 128, 128)], sem)
        for j in range(IDX_ROWS)
    ]
    for c in copies:
        c.wait()
    pltpu.sync_copy(rows_v, raw_hbm.at[pl.ds(base, BPW)])


@functools.cache
def _make_sc_gather():
    return pl.kernel(
        _sc_gather_body,
        out_type=jax.ShapeDtypeStruct((BATCH, NUM_STATES), jnp.int32),
        mesh=plsc.VectorSubcoreMesh(core_axis_name="c", subcore_axis_name="s",
                                    num_cores=NC, num_subcores=NS),
        scratch_types=[
            pltpu.VMEM((IDX_ROWS, 128), jnp.int32),
            pltpu.VMEM((BPW, NUM_STATES), jnp.int32),
            pltpu.SemaphoreType.DMA,
        ],
    )


def _unpack_sub_body(raw_ref, sel_ref, lse_ref, out_ref):
    raw = lax.bitcast_convert_type(raw_ref[...], jnp.uint32)
    sel = sel_ref[...] != 0
    chosen = jnp.where(sel, raw[:, HALF_STATES:], raw[:, :HALF_STATES])
    lo_f = lax.bitcast_convert_type(chosen << 16, jnp.float32)
    hi_f = lax.bitcast_convert_type(chosen & jnp.uint32(0xFFFF0000),
                                    jnp.float32)
    out_ref[...] = jnp.concatenate([lo_f, hi_f], axis=1) - lse_ref[...]


_epilogue = pl.pallas_call(
    _unpack_sub_body,
    grid=(EPI_NBLK,),
    in_specs=[
        pl.BlockSpec((EPI_BLK, NUM_STATES), lambda i: (i, 0)),
        pl.BlockSpec((EPI_BLK, 1), lambda i: (i, 0)),
        pl.BlockSpec((1, NUM_STATES), lambda i: (0, 0)),
    ],
    out_specs=pl.BlockSpec((EPI_BLK, NUM_STATES), lambda i: (i, 0)),
    out_shape=jax.ShapeDtypeStruct((BATCH, NUM_STATES), jnp.float32),
    compiler_params=pltpu.CompilerParams(
        dimension_semantics=("arbitrary",)),
)


def kernel(emission_logits_matrix, observation):
    obs = observation.astype(jnp.int32)
    k = obs % COL_BLK
    idx_row = ((obs // COL_BLK) * HALF_BLK + k % HALF_BLK).reshape(
        NW, IDX_ROWS, 128)
    sel = (k // HALF_BLK).reshape(BATCH, 1)
    tpk, lse = _tc_pass(emission_logits_matrix)
    raw = _make_sc_gather()(tpk, idx_row)
    return _epilogue(raw, sel, lse)


# speedup vs baseline: 1.1852x; 1.1852x over previous
import functools

import jax
import jax.numpy as jnp
from jax import lax
from jax.experimental import pallas as pl
from jax.experimental.pallas import tpu as pltpu
from jax.experimental.pallas import tpu_sc as plsc

NUM_STATES = 128
HALF_STATES = NUM_STATES // 2
NUM_OBS = 100000
BATCH = 16384

COL_BLK = 8192
HALF_BLK = COL_BLK // 2
NBLK = (NUM_OBS + COL_BLK - 1) // COL_BLK
TBL_ROWS = NBLK * HALF_BLK

NC = 2
NS = 16
NW = NC * NS
BPW = BATCH // NW
IDX_ROWS = BPW // 128

EPI_BLK = 2048
EPI_NBLK = BATCH // EPI_BLK


def _round_bits(x):
    b = lax.bitcast_convert_type(x, jnp.uint32)
    return (b + jnp.uint32(0x8000)) & jnp.uint32(0xFFFF0000)


def _lse_transpose_pack_body(x_ref, tpk_ref, lse_ref, m_acc, s_acc):
    i = pl.program_id(0)
    x = x_ref[...]

    lo_bits = _round_bits(x[:, :HALF_BLK]) >> 16
    hi_bits = _round_bits(x[:, HALF_BLK:])
    tpk_ref[...] = lax.bitcast_convert_type((lo_bits | hi_bits).T, jnp.int32)

    @pl.when(i == 0)
    def _():
        m_acc[...] = jnp.full((NUM_STATES, 1), -jnp.inf, jnp.float32)
        s_acc[...] = jnp.zeros((NUM_STATES, 1), jnp.float32)

    def update(x_vals):
        m_old = m_acc[...]
        m_new = jnp.maximum(m_old, jnp.max(x_vals, axis=1, keepdims=True))
        s_acc[...] = s_acc[...] * jnp.exp(m_old - m_new) + jnp.sum(
            jnp.exp(x_vals - m_new), axis=1, keepdims=True)
        m_acc[...] = m_new

    @pl.when(i < NBLK - 1)
    def _():
        update(x)

    @pl.when(i == NBLK - 1)
    def _():
        col = i * COL_BLK + lax.broadcasted_iota(
            jnp.int32, (NUM_STATES, COL_BLK), 1)
        update(jnp.where(col < NUM_OBS, x, -jnp.inf))
        lse_ref[...] = (m_acc[...] + jnp.log(s_acc[...])).T


_tc_pass = pl.pallas_call(
    _lse_transpose_pack_body,
    grid=(NBLK,),
    in_specs=[pl.BlockSpec((NUM_STATES, COL_BLK), lambda i: (0, i))],
    out_specs=[
        pl.BlockSpec((HALF_BLK, NUM_STATES), lambda i: (i, 0)),
        pl.BlockSpec((1, NUM_STATES), lambda i: (0, 0)),
    ],
    out_shape=[
        jax.ShapeDtypeStruct((TBL_ROWS, NUM_STATES), jnp.int32),
        jax.ShapeDtypeStruct((1, NUM_STATES), jnp.float32),
    ],
    scratch_shapes=[
        pltpu.VMEM((NUM_STATES, 1), jnp.float32),
        pltpu.VMEM((NUM_STATES, 1), jnp.float32),
    ],
    compiler_params=pltpu.CompilerParams(
        dimension_semantics=("arbitrary",)),
)


def _sc_gather_body(table_hbm, obs_hbm, raw_hbm, idx_v, rows_v, sem):
    wid = lax.axis_index("s") * NC + lax.axis_index("c")
    base = wid * BPW
    pltpu.sync_copy(obs_hbm.at[wid], idx_v)
    copies = [
        pltpu.async_copy(table_hbm.at[idx_v.at[j]],
                         rows_v.at[pl.ds(j * 128, 128)], sem)
        for j in range(IDX_ROWS)
    ]
    for c in copies:
        c.wait()
    pltpu.sync_copy(rows_v, raw_hbm.at[pl.ds(base, BPW)])


@functools.cache
def _make_sc_gather():
    return pl.kernel(
        _sc_gather_body,
        out_type=jax.ShapeDtypeStruct((BATCH, NUM_STATES), jnp.int32),
        mesh=plsc.VectorSubcoreMesh(core_axis_name="c", subcore_axis_name="s",
                                    num_cores=NC, num_subcores=NS),
        scratch_types=[
            pltpu.VMEM((IDX_ROWS, 128), jnp.int32),
            pltpu.VMEM((BPW, NUM_STATES), jnp.int32),
            pltpu.SemaphoreType.DMA,
        ],
    )


def _unpack_sub_body(raw_ref, sel_ref, lse_ref, out_ref):
    raw = lax.bitcast_convert_type(raw_ref[...], jnp.uint32)
    sel = sel_ref[...] != 0
    lo_f = lax.bitcast_convert_type(raw << 16, jnp.float32)
    hi_f = lax.bitcast_convert_type(raw & jnp.uint32(0xFFFF0000), jnp.float32)
    out_ref[...] = jnp.where(sel, hi_f, lo_f) - lse_ref[...]


_epilogue = pl.pallas_call(
    _unpack_sub_body,
    grid=(EPI_NBLK,),
    in_specs=[
        pl.BlockSpec((EPI_BLK, NUM_STATES), lambda i: (i, 0)),
        pl.BlockSpec((EPI_BLK, 1), lambda i: (i, 0)),
        pl.BlockSpec((1, NUM_STATES), lambda i: (0, 0)),
    ],
    out_specs=pl.BlockSpec((EPI_BLK, NUM_STATES), lambda i: (i, 0)),
    out_shape=jax.ShapeDtypeStruct((BATCH, NUM_STATES), jnp.float32),
    compiler_params=pltpu.CompilerParams(
        dimension_semantics=("arbitrary",)),
)


def kernel(emission_logits_matrix, observation):
    obs = observation.astype(jnp.int32)
    k = obs % COL_BLK
    idx_row = ((obs // COL_BLK) * HALF_BLK + k % HALF_BLK).reshape(
        NW, IDX_ROWS, 128)
    sel = (k // HALF_BLK).reshape(BATCH, 1)
    tpk, lse = _tc_pass(emission_logits_matrix)
    raw = _make_sc_gather()(tpk, idx_row)
    return _epilogue(raw, sel, lse)


# speedup vs baseline: 1.1959x; 1.0090x over previous
import functools

import jax
import jax.numpy as jnp
from jax import lax
from jax.experimental import pallas as pl
from jax.experimental.pallas import tpu as pltpu
from jax.experimental.pallas import tpu_sc as plsc

NUM_STATES = 128
HALF_STATES = NUM_STATES // 2
NUM_OBS = 100000
BATCH = 16384

COL_BLK = 16384
HALF_BLK = COL_BLK // 2
NBLK = (NUM_OBS + COL_BLK - 1) // COL_BLK
TBL_ROWS = NBLK * HALF_BLK

NC = 2
NS = 16
NW = NC * NS
BPW = BATCH // NW
IDX_ROWS = BPW // 128

EPI_BLK = 2048
EPI_NBLK = BATCH // EPI_BLK


def _round_bits(x):
    b = lax.bitcast_convert_type(x, jnp.uint32)
    return (b + jnp.uint32(0x8000)) & jnp.uint32(0xFFFF0000)


def _lse_transpose_pack_body(x_ref, tpk_ref, lse_ref, m_acc, s_acc):
    i = pl.program_id(0)
    x = x_ref[...]

    lo_bits = _round_bits(x[:, :HALF_BLK]) >> 16
    hi_bits = _round_bits(x[:, HALF_BLK:])
    tpk_ref[...] = lax.bitcast_convert_type((lo_bits | hi_bits).T, jnp.int32)

    @pl.when(i == 0)
    def _():
        m_acc[...] = jnp.full((NUM_STATES, 1), -jnp.inf, jnp.float32)
        s_acc[...] = jnp.zeros((NUM_STATES, 1), jnp.float32)

    def update(x_vals):
        m_old = m_acc[...]
        m_new = jnp.maximum(m_old, jnp.max(x_vals, axis=1, keepdims=True))
        s_acc[...] = s_acc[...] * jnp.exp(m_old - m_new) + jnp.sum(
            jnp.exp(x_vals - m_new), axis=1, keepdims=True)
        m_acc[...] = m_new

    @pl.when(i < NBLK - 1)
    def _():
        update(x)

    @pl.when(i == NBLK - 1)
    def _():
        col = i * COL_BLK + lax.broadcasted_iota(
            jnp.int32, (NUM_STATES, COL_BLK), 1)
        update(jnp.where(col < NUM_OBS, x, -jnp.inf))
        lse_ref[...] = (m_acc[...] + jnp.log(s_acc[...])).T


_tc_pass = pl.pallas_call(
    _lse_transpose_pack_body,
    grid=(NBLK,),
    in_specs=[pl.BlockSpec((NUM_STATES, COL_BLK), lambda i: (0, i))],
    out_specs=[
        pl.BlockSpec((HALF_BLK, NUM_STATES), lambda i: (i, 0)),
        pl.BlockSpec((1, NUM_STATES), lambda i: (0, 0)),
    ],
    out_shape=[
        jax.ShapeDtypeStruct((TBL_ROWS, NUM_STATES), jnp.int32),
        jax.ShapeDtypeStruct((1, NUM_STATES), jnp.float32),
    ],
    scratch_shapes=[
        pltpu.VMEM((NUM_STATES, 1), jnp.float32),
        pltpu.VMEM((NUM_STATES, 1), jnp.float32),
    ],
    compiler_params=pltpu.CompilerParams(
        dimension_semantics=("arbitrary",)),
)


def _sc_gather_body(table_hbm, obs_hbm, raw_hbm, idx_v, rows_v, sem):
    wid = lax.axis_index("s") * NC + lax.axis_index("c")
    base = wid * BPW
    pltpu.sync_copy(obs_hbm.at[wid], idx_v)
    copies = [
        pltpu.async_copy(table_hbm.at[idx_v.at[j]],
                         rows_v.at[pl.ds(j * 128, 128)], sem)
        for j in range(IDX_ROWS)
    ]
    for c in copies:
        c.wait()
    pltpu.sync_copy(rows_v, raw_hbm.at[pl.ds(base, BPW)])


@functools.cache
def _make_sc_gather():
    return pl.kernel(
        _sc_gather_body,
        out_type=jax.ShapeDtypeStruct((BATCH, NUM_STATES), jnp.int32),
        mesh=plsc.VectorSubcoreMesh(core_axis_name="c", subcore_axis_name="s",
                                    num_cores=NC, num_subcores=NS),
        scratch_types=[
            pltpu.VMEM((IDX_ROWS, 128), jnp.int32),
            pltpu.VMEM((BPW, NUM_STATES), jnp.int32),
            pltpu.SemaphoreType.DMA,
        ],
    )


def _unpack_sub_body(raw_ref, sel_ref, lse_ref, out_ref):
    raw = lax.bitcast_convert_type(raw_ref[...], jnp.uint32)
    sel = sel_ref[...] != 0
    lo_f = lax.bitcast_convert_type(raw << 16, jnp.float32)
    hi_f = lax.bitcast_convert_type(raw & jnp.uint32(0xFFFF0000), jnp.float32)
    out_ref[...] = jnp.where(sel, hi_f, lo_f) - lse_ref[...]


_epilogue = pl.pallas_call(
    _unpack_sub_body,
    grid=(EPI_NBLK,),
    in_specs=[
        pl.BlockSpec((EPI_BLK, NUM_STATES), lambda i: (i, 0)),
        pl.BlockSpec((EPI_BLK, 1), lambda i: (i, 0)),
        pl.BlockSpec((1, NUM_STATES), lambda i: (0, 0)),
    ],
    out_specs=pl.BlockSpec((EPI_BLK, NUM_STATES), lambda i: (i, 0)),
    out_shape=jax.ShapeDtypeStruct((BATCH, NUM_STATES), jnp.float32),
    compiler_params=pltpu.CompilerParams(
        dimension_semantics=("arbitrary",)),
)


def kernel(emission_logits_matrix, observation):
    obs = observation.astype(jnp.int32)
    k = obs % COL_BLK
    idx_row = ((obs // COL_BLK) * HALF_BLK + k % HALF_BLK).reshape(
        NW, IDX_ROWS, 128)
    sel = (k // HALF_BLK).reshape(BATCH, 1)
    tpk, lse = _tc_pass(emission_logits_matrix)
    raw = _make_sc_gather()(tpk, idx_row)
    return _epilogue(raw, sel, lse)
